# Initial kernel scaffold; baseline (speedup 1.0000x reference)
import jax
import jax.numpy as jnp
from jax.experimental import pallas as pl


def kernel(x, di):
    raise NotImplementedError("write your pallas kernel here")


# speedup vs baseline: 1.4764x; 1.4764x over previous
import functools

import jax
import jax.numpy as jnp
from jax import lax
from jax.experimental import pallas as pl
from jax.experimental.pallas import tpu as pltpu
from jax.experimental.pallas import tpu_sc as plsc

_V = 1000
_D = 252
_DP = 256
_B = 4096 * 26
_NW = 32
_BPW = _B // _NW
_K = 128
_NCHUNK = _BPW // _K

_mesh = plsc.VectorSubcoreMesh(core_axis_name="c", subcore_axis_name="s")


@functools.partial(
    pl.kernel,
    out_type=jax.ShapeDtypeStruct((_B, _D), jnp.float32),
    mesh=_mesh,
    compiler_params=pltpu.CompilerParams(use_tc_tiling_on_sc=False),
    scratch_types=[
        pltpu.VMEM((_K,), jnp.int32),
        pltpu.VMEM((_K, _DP), jnp.float32),
        pltpu.VMEM((_K, _D), jnp.float32),
        pltpu.SemaphoreType.DMA,
    ],
)
def _gather_sc(x_hbm, di_hbm, out_hbm, idx_v, buf, cbuf, sem):
    wid = lax.axis_index("s") * 2 + lax.axis_index("c")
    base = wid * _BPW

    def chunk_body(c, carry):
        cb = base + c * _K
        pltpu.sync_copy(x_hbm.at[pl.ds(cb, _K)], idx_v)
        pltpu.async_copy(di_hbm.at[idx_v], buf, sem).wait()

        def row_body(r, rcarry):
            for k in range(15):
                cbuf[r, pl.ds(16 * k, 16)] = buf[r, pl.ds(16 * k, 16)]
            cbuf[r, pl.ds(_D - 16, 16)] = buf[r, pl.ds(_D - 16, 16)]
            return rcarry

        lax.fori_loop(0, _K, row_body, 0)
        pltpu.sync_copy(cbuf, out_hbm.at[pl.ds(cb, _K)])
        return carry

    lax.fori_loop(0, _NCHUNK, chunk_body, 0)


def kernel(x, di):
    x_flat = x.reshape(-1).astype(jnp.int32)
    di_pad = jnp.pad(di, ((0, 0), (0, _DP - _D)))
    out = _gather_sc(x_flat, di_pad)
    return out.reshape(x.shape[0], x.shape[1], _D)


# speedup vs baseline: 1.5951x; 1.0804x over previous
import functools

import jax
import jax.numpy as jnp
from jax import lax
from jax.experimental import pallas as pl
from jax.experimental.pallas import tpu as pltpu
from jax.experimental.pallas import tpu_sc as plsc

_V = 1000
_D = 252
_DP = 256
_B = 4096 * 26
_NW = 32
_BPW = _B // _NW
_K = 64
_NCHUNK = _BPW // _K
_NPAIR = _NCHUNK // 2

_mesh = plsc.VectorSubcoreMesh(core_axis_name="c", subcore_axis_name="s")


@functools.partial(
    pl.kernel,
    out_type=jax.ShapeDtypeStruct((_B, _D), jnp.float32),
    mesh=_mesh,
    compiler_params=pltpu.CompilerParams(use_tc_tiling_on_sc=False),
    scratch_types=[
        pltpu.VMEM((_BPW,), jnp.int32),
        pltpu.VMEM((_K, _DP), jnp.float32),
        pltpu.VMEM((_K, _DP), jnp.float32),
        pltpu.VMEM((_K, _D), jnp.float32),
        pltpu.VMEM((_K, _D), jnp.float32),
        pltpu.SemaphoreType.DMA,
        pltpu.SemaphoreType.DMA,
        pltpu.SemaphoreType.DMA,
        pltpu.SemaphoreType.DMA,
    ],
)
def _gather_sc(x_hbm, di_hbm, out_hbm, idx_v, buf0, buf1, cbuf0, cbuf1,
               sg0, sg1, sw0, sw1):
    wid = lax.axis_index("s") * 2 + lax.axis_index("c")
    base = wid * _BPW
    pltpu.sync_copy(x_hbm.at[pl.ds(base, _BPW)], idx_v)

    def idx_of(c):
        return idx_v.at[pl.ds(c * _K, _K)]

    def out_of(c):
        return out_hbm.at[pl.ds(base + c * _K, _K)]

    def compact(buf, cbuf):
        def row_body(r, rcarry):
            for k in range(15):
                cbuf[r, pl.ds(16 * k, 16)] = buf[r, pl.ds(16 * k, 16)]
            cbuf[r, pl.ds(_D - 16, 16)] = buf[r, pl.ds(_D - 16, 16)]
            return rcarry

        lax.fori_loop(0, _K, row_body, 0)

    pltpu.async_copy(di_hbm.at[idx_of(0)], buf0, sg0)

    def pair_body(h, carry):
        c0 = 2 * h

        pltpu.make_async_copy(di_hbm.at[idx_of(c0)], buf0, sg0).wait()
        pltpu.async_copy(di_hbm.at[idx_of(c0 + 1)], buf1, sg1)

        @pl.when(h > 0)
        def _w0():
            pltpu.make_async_copy(cbuf0, out_of(c0 - 2), sw0).wait()

        compact(buf0, cbuf0)
        pltpu.async_copy(cbuf0, out_of(c0), sw0)

        pltpu.make_async_copy(di_hbm.at[idx_of(c0 + 1)], buf1, sg1).wait()

        @pl.when(h < _NPAIR - 1)
        def _g0():
            pltpu.async_copy(di_hbm.at[idx_of(c0 + 2)], buf0, sg0)

        @pl.when(h > 0)
        def _w1():
            pltpu.make_async_copy(cbuf1, out_of(c0 - 1), sw1).wait()

        compact(buf1, cbuf1)
        pltpu.async_copy(cbuf1, out_of(c0 + 1), sw1)
        return carry

    lax.fori_loop(0, _NPAIR, pair_body, 0)
    pltpu.make_async_copy(cbuf0, out_of(_NCHUNK - 2), sw0).wait()
    pltpu.make_async_copy(cbuf1, out_of(_NCHUNK - 1), sw1).wait()


def kernel(x, di):
    x_flat = x.reshape(-1).astype(jnp.int32)
    di_pad = jnp.pad(di, ((0, 0), (0, _DP - _D)))
    out = _gather_sc(x_flat, di_pad)
    return out.reshape(x.shape[0], x.shape[1], _D)


# speedup vs baseline: 2.6199x; 1.6425x over previous
import functools

import jax
import jax.numpy as jnp
from jax import lax
from jax.experimental import pallas as pl
from jax.experimental.pallas import tpu as pltpu
from jax.experimental.pallas import tpu_sc as plsc

_V = 1000
_D = 252
_DP = 256
_NI = 4096
_NJ = 26
_NW = 32
_IPW = _NI // _NW
_CI = 4
_K = _CI * _NJ
_NCHUNK = _IPW // _CI
_NPAIR = _NCHUNK // 2
_BPW = _IPW * _NJ

_mesh = plsc.VectorSubcoreMesh(core_axis_name="c", subcore_axis_name="s")


@functools.partial(
    pl.kernel,
    out_type=jax.ShapeDtypeStruct((_NI, _NJ, _D), jnp.float32),
    mesh=_mesh,
    scratch_types=[
        pltpu.VMEM((_BPW,), jnp.int32),
        pltpu.VMEM((_K, _DP), jnp.float32),
        pltpu.VMEM((_K, _DP), jnp.float32),
        pltpu.VMEM((_CI, _NJ, _D), jnp.float32),
        pltpu.VMEM((_CI, _NJ, _D), jnp.float32),
        pltpu.SemaphoreType.DMA,
        pltpu.SemaphoreType.DMA,
        pltpu.SemaphoreType.DMA,
        pltpu.SemaphoreType.DMA,
    ],
)
def _gather_sc(x_hbm, di_hbm, out_hbm, idx_v, buf0, buf1, cbuf0, cbuf1,
               sg0, sg1, sw0, sw1):
    wid = lax.axis_index("s") * 2 + lax.axis_index("c")
    i_base = wid * _IPW
    pltpu.sync_copy(x_hbm.at[pl.ds(i_base * _NJ, _BPW)], idx_v)

    def idx_of(c):
        return idx_v.at[pl.ds(c * _K, _K)]

    def out_of(c):
        return out_hbm.at[pl.ds(i_base + c * _CI, _CI)]

    def compact(buf, cbuf):
        for a in range(_CI):
            def row_body(j, jcarry, a=a):
                r = a * _NJ + j
                for k in range(15):
                    cbuf[a, j, pl.ds(16 * k, 16)] = buf[r, pl.ds(16 * k, 16)]
                cbuf[a, j, pl.ds(_D - 16, 16)] = buf[r, pl.ds(_D - 16, 16)]
                return jcarry

            lax.fori_loop(0, _NJ, row_body, 0)

    pltpu.async_copy(di_hbm.at[idx_of(0)], buf0, sg0)

    def pair_body(h, carry):
        c0 = 2 * h

        pltpu.make_async_copy(di_hbm.at[idx_of(c0)], buf0, sg0).wait()
        pltpu.async_copy(di_hbm.at[idx_of(c0 + 1)], buf1, sg1)

        @pl.when(h > 0)
        def _w0():
            pltpu.make_async_copy(cbuf0, out_of(c0 - 2), sw0).wait()

        compact(buf0, cbuf0)
        pltpu.async_copy(cbuf0, out_of(c0), sw0)

        pltpu.make_async_copy(di_hbm.at[idx_of(c0 + 1)], buf1, sg1).wait()

        @pl.when(h < _NPAIR - 1)
        def _g0():
            pltpu.async_copy(di_hbm.at[idx_of(c0 + 2)], buf0, sg0)

        @pl.when(h > 0)
        def _w1():
            pltpu.make_async_copy(cbuf1, out_of(c0 - 1), sw1).wait()

        compact(buf1, cbuf1)
        pltpu.async_copy(cbuf1, out_of(c0 + 1), sw1)
        return carry

    lax.fori_loop(0, _NPAIR, pair_body, 0)
    pltpu.make_async_copy(cbuf0, out_of(_NCHUNK - 2), sw0).wait()
    pltpu.make_async_copy(cbuf1, out_of(_NCHUNK - 1), sw1).wait()


def kernel(x, di):
    x_flat = x.reshape(-1).astype(jnp.int32)
    di_pad = jnp.pad(di, ((0, 0), (0, _DP - _D)))
    return _gather_sc(x_flat, di_pad)


# speedup vs baseline: 5.4883x; 2.0948x over previous
import functools

import jax
import jax.numpy as jnp
from jax import lax
from jax.experimental import pallas as pl
from jax.experimental.pallas import tpu as pltpu
from jax.experimental.pallas import tpu_sc as plsc

_V = 1000
_D = 252
_DP = 256
_NI = 4096
_NJ = 26
_NW = 32
_IPW = _NI // _NW
_K = 64
_HALF = _IPW // _K
_NCHUNK = _NJ * _HALF
_NPAIR = _NCHUNK // 2

_mesh = plsc.VectorSubcoreMesh(core_axis_name="c", subcore_axis_name="s")


@functools.partial(
    pl.kernel,
    out_type=jax.ShapeDtypeStruct((_NJ, _NI, _D), jnp.float32),
    mesh=_mesh,
    scratch_types=[
        pltpu.VMEM((_NJ, _IPW), jnp.int32),
        pltpu.VMEM((_K, _DP), jnp.float32),
        pltpu.VMEM((_K, _DP), jnp.float32),
        pltpu.VMEM((_K, _D), jnp.float32),
        pltpu.VMEM((_K, _D), jnp.float32),
        pltpu.SemaphoreType.DMA,
        pltpu.SemaphoreType.DMA,
        pltpu.SemaphoreType.DMA,
        pltpu.SemaphoreType.DMA,
    ],
)
def _gather_sc(xt_hbm, di_hbm, out_hbm, idx_v, buf0, buf1, cbuf0, cbuf1,
               sg0, sg1, sw0, sw1):
    wid = lax.axis_index("s") * 2 + lax.axis_index("c")
    i_base = wid * _IPW
    pltpu.sync_copy(xt_hbm.at[:, pl.ds(i_base, _IPW)], idx_v)

    def idx_of(c):
        j = c // _HALF
        h = c % _HALF
        return idx_v.at[j, pl.ds(h * _K, _K)]

    def out_of(c):
        j = c // _HALF
        h = c % _HALF
        return out_hbm.at[j, pl.ds(i_base + h * _K, _K)]

    def compact(buf, cbuf):
        def row_body(r, rcarry):
            for k in range(15):
                cbuf[r, pl.ds(16 * k, 16)] = buf[r, pl.ds(16 * k, 16)]
            cbuf[r, pl.ds(_D - 16, 16)] = buf[r, pl.ds(_D - 16, 16)]
            return rcarry

        lax.fori_loop(0, _K, row_body, 0)

    pltpu.async_copy(di_hbm.at[idx_of(0)], buf0, sg0)

    def pair_body(h, carry):
        c0 = 2 * h

        pltpu.make_async_copy(di_hbm.at[idx_of(c0)], buf0, sg0).wait()
        pltpu.async_copy(di_hbm.at[idx_of(c0 + 1)], buf1, sg1)

        @pl.when(h > 0)
        def _w0():
            pltpu.make_async_copy(cbuf0, out_of(c0 - 2), sw0).wait()

        compact(buf0, cbuf0)
        pltpu.async_copy(cbuf0, out_of(c0), sw0)

        pltpu.make_async_copy(di_hbm.at[idx_of(c0 + 1)], buf1, sg1).wait()

        @pl.when(h < _NPAIR - 1)
        def _g0():
            pltpu.async_copy(di_hbm.at[idx_of(c0 + 2)], buf0, sg0)

        @pl.when(h > 0)
        def _w1():
            pltpu.make_async_copy(cbuf1, out_of(c0 - 1), sw1).wait()

        compact(buf1, cbuf1)
        pltpu.async_copy(cbuf1, out_of(c0 + 1), sw1)
        return carry

    lax.fori_loop(0, _NPAIR, pair_body, 0)
    pltpu.make_async_copy(cbuf0, out_of(_NCHUNK - 2), sw0).wait()
    pltpu.make_async_copy(cbuf1, out_of(_NCHUNK - 1), sw1).wait()


def kernel(x, di):
    xt = x.T.astype(jnp.int32)
    di_pad = jnp.pad(di, ((0, 0), (0, _DP - _D)))
    out = _gather_sc(xt, di_pad)
    return out.transpose(1, 0, 2)


# speedup vs baseline: 5.8970x; 1.0745x over previous
import functools

import jax
import jax.numpy as jnp
from jax import lax
from jax.experimental import pallas as pl
from jax.experimental.pallas import tpu as pltpu
from jax.experimental.pallas import tpu_sc as plsc

_V = 1000
_D = 252
_DP = 256
_NI = 4096
_NJ = 26
_NW = 32
_IPW = _NI // _NW
_KH = _IPW // 2
_NPAIR = _NJ // 2

_mesh = plsc.VectorSubcoreMesh(core_axis_name="c", subcore_axis_name="s")


@functools.partial(
    pl.kernel,
    out_type=jax.ShapeDtypeStruct((_NJ, _NI, _D), jnp.float32),
    mesh=_mesh,
    scratch_types=[
        pltpu.VMEM((_NJ, _IPW), jnp.int32),
        pltpu.VMEM((_IPW, _DP), jnp.float32),
        pltpu.VMEM((_IPW, _DP), jnp.float32),
        pltpu.VMEM((_KH, _D), jnp.float32),
        pltpu.VMEM((_KH, _D), jnp.float32),
        pltpu.SemaphoreType.DMA,
        pltpu.SemaphoreType.DMA,
        pltpu.SemaphoreType.DMA,
        pltpu.SemaphoreType.DMA,
    ],
)
def _gather_sc(xt_hbm, di_hbm, out_hbm, idx_v, buf0, buf1, cbuf0, cbuf1,
               sg0, sg1, sw0, sw1):
    wid = lax.axis_index("s") * 2 + lax.axis_index("c")
    i_base = wid * _IPW
    pltpu.sync_copy(xt_hbm.at[:, pl.ds(i_base, _IPW)], idx_v)

    def idx_of(j):
        return idx_v.at[j]

    def out_of(j, s):
        return out_hbm.at[j, pl.ds(i_base + s * _KH, _KH)]

    def compact_half(buf, cbuf, s):
        def row_body(r, rcarry):
            for k in range(15):
                cbuf[r, pl.ds(16 * k, 16)] = buf[s * _KH + r, pl.ds(16 * k, 16)]
            cbuf[r, pl.ds(_D - 16, 16)] = buf[s * _KH + r, pl.ds(_D - 16, 16)]
            return rcarry

        lax.fori_loop(0, _KH, row_body, 0)

    def process(buf, j, first):
        @pl.when(jnp.logical_not(first))
        def _wait_w0():
            pltpu.make_async_copy(cbuf0, out_of(j - 1, 0), sw0).wait()

        compact_half(buf, cbuf0, 0)
        pltpu.async_copy(cbuf0, out_of(j, 0), sw0)

        @pl.when(jnp.logical_not(first))
        def _wait_w1():
            pltpu.make_async_copy(cbuf1, out_of(j - 1, 1), sw1).wait()

        compact_half(buf, cbuf1, 1)
        pltpu.async_copy(cbuf1, out_of(j, 1), sw1)

    pltpu.async_copy(di_hbm.at[idx_of(0)], buf0, sg0)

    def pair_body(h, carry):
        j0 = 2 * h

        pltpu.make_async_copy(di_hbm.at[idx_of(j0)], buf0, sg0).wait()
        pltpu.async_copy(di_hbm.at[idx_of(j0 + 1)], buf1, sg1)
        process(buf0, j0, h == 0)

        pltpu.make_async_copy(di_hbm.at[idx_of(j0 + 1)], buf1, sg1).wait()

        @pl.when(h < _NPAIR - 1)
        def _g0():
            pltpu.async_copy(di_hbm.at[idx_of(j0 + 2)], buf0, sg0)

        process(buf1, j0 + 1, False)
        return carry

    lax.fori_loop(0, _NPAIR, pair_body, 0)
    pltpu.make_async_copy(cbuf0, out_of(_NJ - 1, 0), sw0).wait()
    pltpu.make_async_copy(cbuf1, out_of(_NJ - 1, 1), sw1).wait()


def kernel(x, di):
    xt = x.T.astype(jnp.int32)
    di_pad = jnp.pad(di, ((0, 0), (0, _DP - _D)))
    out = _gather_sc(xt, di_pad)
    return out.transpose(1, 0, 2)
